# baseline (speedup 1.0000x reference)
import functools

import jax
import jax.numpy as jnp
from jax import lax
from jax.experimental import pallas as pl
from jax.experimental.pallas import tpu as pltpu
from jax.experimental.pallas import tpu_sc as plsc

H = 1024
F = 2048
E = 8
T = 2048
BLK = 256
MAXBLKS = T * 2 // BLK + E
NPAD = MAXBLKS * BLK
NW = 32
TPW = T // NW


def _router_body(x_ref, rw_ref, rb_ref, pos0_ref, pos1_ref, g0_ref, g1_ref,
                 meta_ref, xb_ref):
    xf = x_ref[...]
    xb_ref[...] = xf.astype(jnp.bfloat16)
    logits = jnp.dot(xf, rw_ref[...], preferred_element_type=jnp.float32) + rb_ref[...]
    m = jnp.max(logits, axis=-1, keepdims=True)
    ex = jnp.exp(logits - m)
    probs = ex / jnp.sum(ex, axis=-1, keepdims=True)

    iota_e = lax.broadcasted_iota(jnp.int32, (T, E), 1)
    p0 = jnp.max(probs, axis=-1, keepdims=True)
    ids0 = jnp.min(jnp.where(probs == p0, iota_e, E), axis=-1, keepdims=True)
    sel0 = iota_e == ids0
    masked = jnp.where(sel0, -1e30, probs)
    p1 = jnp.max(masked, axis=-1, keepdims=True)
    ids1 = jnp.min(jnp.where(masked == p1, iota_e, E), axis=-1, keepdims=True)
    sel1 = iota_e == ids1
    denom = p0 + p1
    g0_ref[...] = p0 / denom
    g1_ref[...] = p1 / denom

    c0 = sel0.astype(jnp.float32)
    c1 = sel1.astype(jnp.float32)
    cnt = c0 + c1
    r_i = lax.broadcasted_iota(jnp.int32, (T, T), 0)
    c_i = lax.broadcasted_iota(jnp.int32, (T, T), 1)
    tri = (r_i > c_i).astype(jnp.float32)
    cum_before = jnp.dot(tri, cnt, preferred_element_type=jnp.float32)
    counts = jnp.sum(cnt, axis=0, keepdims=True)
    nblk = jnp.floor((counts + (BLK - 1)) * (1.0 / BLK))
    u_r = lax.broadcasted_iota(jnp.int32, (E, E), 0)
    u_c = lax.broadcasted_iota(jnp.int32, (E, E), 1)
    ut = (u_r <= u_c).astype(jnp.float32)
    s_incl = jnp.dot(nblk, ut, preferred_element_type=jnp.float32)
    pad_off = BLK * (s_incl - nblk)

    pos0f = jnp.sum(jnp.where(sel0, cum_before + pad_off, 0.0), axis=-1, keepdims=True)
    pos1f = jnp.sum(jnp.where(sel1, cum_before + c0 + pad_off, 0.0), axis=-1, keepdims=True)
    pos0_ref[...] = pos0f.astype(jnp.int32)
    pos1_ref[...] = pos1f.astype(jnp.int32)

    iota_b = lax.broadcasted_iota(jnp.int32, (1, 32), 1)
    iota_bf = iota_b.astype(jnp.float32)
    bef = jnp.zeros((1, 32), jnp.float32)
    for e in range(E):
        bef = bef + (s_incl[:, e:e + 1] <= iota_bf).astype(jnp.float32)
    be = jnp.minimum(bef, float(E - 1)).astype(jnp.int32)
    nact = s_incl[:, E - 1:E].astype(jnp.int32)
    meta_ref[...] = jnp.where(iota_b == MAXBLKS, nact, be)


def _router(xf, rw, rb2):
    return pl.pallas_call(
        _router_body,
        out_shape=[
            jax.ShapeDtypeStruct((T, 1), jnp.int32),
            jax.ShapeDtypeStruct((T, 1), jnp.int32),
            jax.ShapeDtypeStruct((T, 1), jnp.float32),
            jax.ShapeDtypeStruct((T, 1), jnp.float32),
            jax.ShapeDtypeStruct((1, 32), jnp.int32),
            jax.ShapeDtypeStruct((T, H), jnp.bfloat16),
        ],
    )(xf, rw, rb2)


def _dispatch_body(x_hbm, p0_hbm, p1_hbm, g0_hbm, g1_hbm, xs_hbm, gs_hbm,
                   rows_v, idx0_v, idx1_v, g0_v, g1_v, sem0, sem1, sem2, sem3):
    wid = lax.axis_index("s") * 2 + lax.axis_index("c")
    base = wid * TPW
    pltpu.sync_copy(x_hbm.at[pl.ds(base, TPW)], rows_v)
    pltpu.sync_copy(p0_hbm.at[pl.ds(base, TPW)], idx0_v)
    pltpu.sync_copy(p1_hbm.at[pl.ds(base, TPW)], idx1_v)
    pltpu.sync_copy(g0_hbm.at[pl.ds(base, TPW)], g0_v)
    pltpu.sync_copy(g1_hbm.at[pl.ds(base, TPW)], g1_v)
    c0 = pltpu.async_copy(rows_v, xs_hbm.at[idx0_v], sem0)
    c1 = pltpu.async_copy(rows_v, xs_hbm.at[idx1_v], sem1)
    c2 = pltpu.async_copy(g0_v, gs_hbm.at[idx0_v], sem2)
    c3 = pltpu.async_copy(g1_v, gs_hbm.at[idx1_v], sem3)
    c0.wait()
    c1.wait()
    c2.wait()
    c3.wait()


@functools.partial(
    pl.kernel,
    out_type=[
        jax.ShapeDtypeStruct((NPAD, H // 2), jnp.int32),
        jax.ShapeDtypeStruct((NPAD,), jnp.float32),
    ],
    mesh=plsc.VectorSubcoreMesh(core_axis_name="c", subcore_axis_name="s"),
    scratch_types=[
        pltpu.VMEM((TPW, H // 2), jnp.int32),
        pltpu.VMEM((TPW,), jnp.int32),
        pltpu.VMEM((TPW,), jnp.int32),
        pltpu.VMEM((TPW,), jnp.float32),
        pltpu.VMEM((TPW,), jnp.float32),
        pltpu.SemaphoreType.DMA,
        pltpu.SemaphoreType.DMA,
        pltpu.SemaphoreType.DMA,
        pltpu.SemaphoreType.DMA,
    ],
)
def _dispatch(x_hbm, p0_hbm, p1_hbm, g0_hbm, g1_hbm, xs_hbm, gs_hbm,
              rows_v, idx0_v, idx1_v, g0_v, g1_v, sem0, sem1, sem2, sem3):
    _dispatch_body(x_hbm, p0_hbm, p1_hbm, g0_hbm, g1_hbm, xs_hbm, gs_hbm,
                   rows_v, idx0_v, idx1_v, g0_v, g1_v, sem0, sem1, sem2, sem3)


def _gemm_body(s_ref, xs_ref, w1_ref, b1_ref, w2_ref, b2_ref, gs_ref, y_ref):
    b = pl.program_id(0)

    @pl.when(b < s_ref[MAXBLKS])
    def _():
        h = jnp.dot(xs_ref[...], w1_ref[...], preferred_element_type=jnp.float32)
        h = jnp.maximum(h + b1_ref[...], 0.0).astype(jnp.bfloat16)
        y = jnp.dot(h, w2_ref[...], preferred_element_type=jnp.float32) + b2_ref[...]
        y_ref[...] = y * gs_ref[...]


def _gemm(meta, xs, w1, b1r, w2, b2r, gs2):
    grid_spec = pltpu.PrefetchScalarGridSpec(
        num_scalar_prefetch=1,
        grid=(MAXBLKS,),
        in_specs=[
            pl.BlockSpec((BLK, H), lambda b, s: (b, 0)),
            pl.BlockSpec((None, H, F), lambda b, s: (s[b], 0, 0)),
            pl.BlockSpec((None, 1, F), lambda b, s: (s[b], 0, 0)),
            pl.BlockSpec((None, F, H), lambda b, s: (s[b], 0, 0)),
            pl.BlockSpec((None, 1, H), lambda b, s: (s[b], 0, 0)),
            pl.BlockSpec((BLK, 1), lambda b, s: (b, 0)),
        ],
        out_specs=pl.BlockSpec((BLK, H), lambda b, s: (b, 0)),
    )
    return pl.pallas_call(
        _gemm_body,
        grid_spec=grid_spec,
        out_shape=jax.ShapeDtypeStruct((NPAD, H), jnp.float32),
        compiler_params=pltpu.CompilerParams(dimension_semantics=("arbitrary",)),
    )(meta, xs, w1, b1r, w2, b2r, gs2)


def _combine_body(ys_hbm, p0_hbm, p1_hbm, out_hbm, y0_v, y1_v, idx0_v, idx1_v,
                  sem0, sem1):
    wid = lax.axis_index("s") * 2 + lax.axis_index("c")
    for c in range(2):
        base = wid * TPW + c * (TPW // 2)
        pltpu.sync_copy(p0_hbm.at[pl.ds(base, TPW // 2)], idx0_v)
        pltpu.sync_copy(p1_hbm.at[pl.ds(base, TPW // 2)], idx1_v)
        c0 = pltpu.async_copy(ys_hbm.at[idx0_v], y0_v, sem0)
        c1 = pltpu.async_copy(ys_hbm.at[idx1_v], y1_v, sem1)
        c0.wait()
        c1.wait()

        def tbody(t, carry):
            def jbody(j, carry2):
                sl = pl.ds(j * 16, 16)
                y0_v[t, sl] = y0_v[t, sl] + y1_v[t, sl]
                return carry2
            return lax.fori_loop(0, H // 16, jbody, carry)
        lax.fori_loop(0, TPW // 2, tbody, 0)
        pltpu.sync_copy(y0_v, out_hbm.at[pl.ds(base, TPW // 2)])


@functools.partial(
    pl.kernel,
    out_type=jax.ShapeDtypeStruct((T, H), jnp.float32),
    mesh=plsc.VectorSubcoreMesh(core_axis_name="c", subcore_axis_name="s"),
    scratch_types=[
        pltpu.VMEM((TPW // 2, H), jnp.float32),
        pltpu.VMEM((TPW // 2, H), jnp.float32),
        pltpu.VMEM((TPW // 2,), jnp.int32),
        pltpu.VMEM((TPW // 2,), jnp.int32),
        pltpu.SemaphoreType.DMA,
        pltpu.SemaphoreType.DMA,
    ],
)
def _combine(ys_hbm, p0_hbm, p1_hbm, out_hbm, y0_v, y1_v, idx0_v, idx1_v,
             sem0, sem1):
    _combine_body(ys_hbm, p0_hbm, p1_hbm, out_hbm, y0_v, y1_v, idx0_v, idx1_v,
                  sem0, sem1)


def kernel(x, router_w, router_b, w1, b1, w2, b2):
    batch, seq, hidden = x.shape
    xf = x.reshape(T, H)
    pos0, pos1, g0, g1, meta, xb = _router(xf, router_w, router_b.reshape(1, E))
    pos0 = pos0.reshape(T)
    pos1 = pos1.reshape(T)
    xb32 = jax.lax.bitcast_convert_type(xb.reshape(T, H // 2, 2), jnp.int32)
    xs32, gs = _dispatch(xb32, pos0, pos1, g0.reshape(T), g1.reshape(T))
    xs = jax.lax.bitcast_convert_type(xs32, jnp.bfloat16).reshape(NPAD, H)
    ys = _gemm(meta.reshape(32), xs, w1.astype(jnp.bfloat16),
               b1.reshape(E, 1, F), w2.astype(jnp.bfloat16),
               b2.reshape(E, 1, H), gs.reshape(NPAD, 1))
    out = _combine(ys, pos0, pos1)
    return out.reshape(batch, seq, hidden)


# speedup vs baseline: 2.1131x; 2.1131x over previous
import functools

import jax
import jax.numpy as jnp
from jax import lax
from jax.experimental import pallas as pl
from jax.experimental.pallas import tpu as pltpu
from jax.experimental.pallas import tpu_sc as plsc

H = 1024
F = 2048
E = 8
T = 2048
BLK = 256
MAXBLKS = T * 2 // BLK + E
NPAD = MAXBLKS * BLK
NW = 32
TPW = T // NW


def _router_body(x_ref, rw_ref, rb_ref, pos0_ref, pos1_ref, g0_ref, g1_ref,
                 meta_ref, xb_ref):
    xf = x_ref[...]
    lo = lax.bitcast_convert_type(
        xf[:, :H // 2].astype(jnp.bfloat16).astype(jnp.float32), jnp.int32)
    hi = lax.bitcast_convert_type(
        xf[:, H // 2:].astype(jnp.bfloat16).astype(jnp.float32), jnp.int32)
    xb_ref[...] = lax.shift_right_logical(lo, 16) | (hi & jnp.int32(-65536))
    logits = jnp.dot(xf, rw_ref[...], preferred_element_type=jnp.float32) + rb_ref[...]
    m = jnp.max(logits, axis=-1, keepdims=True)
    ex = jnp.exp(logits - m)
    probs = ex / jnp.sum(ex, axis=-1, keepdims=True)

    iota_e = lax.broadcasted_iota(jnp.int32, (T, E), 1)
    p0 = jnp.max(probs, axis=-1, keepdims=True)
    ids0 = jnp.min(jnp.where(probs == p0, iota_e, E), axis=-1, keepdims=True)
    sel0 = iota_e == ids0
    masked = jnp.where(sel0, -1e30, probs)
    p1 = jnp.max(masked, axis=-1, keepdims=True)
    ids1 = jnp.min(jnp.where(masked == p1, iota_e, E), axis=-1, keepdims=True)
    sel1 = iota_e == ids1
    denom = p0 + p1
    g0_ref[...] = p0 / denom
    g1_ref[...] = p1 / denom

    c0 = sel0.astype(jnp.float32)
    c1 = sel1.astype(jnp.float32)
    cnt = c0 + c1
    r_i = lax.broadcasted_iota(jnp.int32, (T, T), 0)
    c_i = lax.broadcasted_iota(jnp.int32, (T, T), 1)
    tri = (r_i > c_i).astype(jnp.float32)
    cum_before = jnp.dot(tri, cnt, preferred_element_type=jnp.float32)
    counts = jnp.sum(cnt, axis=0, keepdims=True)
    nblk = jnp.floor((counts + (BLK - 1)) * (1.0 / BLK))
    u_r = lax.broadcasted_iota(jnp.int32, (E, E), 0)
    u_c = lax.broadcasted_iota(jnp.int32, (E, E), 1)
    ut = (u_r <= u_c).astype(jnp.float32)
    s_incl = jnp.dot(nblk, ut, preferred_element_type=jnp.float32)
    pad_off = BLK * (s_incl - nblk)

    pos0f = jnp.sum(jnp.where(sel0, cum_before + pad_off, 0.0), axis=-1, keepdims=True)
    pos1f = jnp.sum(jnp.where(sel1, cum_before + c0 + pad_off, 0.0), axis=-1, keepdims=True)
    pos0_ref[...] = pos0f.astype(jnp.int32)
    pos1_ref[...] = pos1f.astype(jnp.int32)

    iota_b = lax.broadcasted_iota(jnp.int32, (1, 32), 1)
    iota_bf = iota_b.astype(jnp.float32)
    bef = jnp.zeros((1, 32), jnp.float32)
    for e in range(E):
        bef = bef + (s_incl[:, e:e + 1] <= iota_bf).astype(jnp.float32)
    be = jnp.minimum(bef, float(E - 1)).astype(jnp.int32)
    nact = s_incl[:, E - 1:E].astype(jnp.int32)
    meta_ref[...] = jnp.where(iota_b == MAXBLKS, nact, be)


def _router(xf, rw, rb2):
    return pl.pallas_call(
        _router_body,
        out_shape=[
            jax.ShapeDtypeStruct((T, 1), jnp.int32),
            jax.ShapeDtypeStruct((T, 1), jnp.int32),
            jax.ShapeDtypeStruct((T, 1), jnp.float32),
            jax.ShapeDtypeStruct((T, 1), jnp.float32),
            jax.ShapeDtypeStruct((1, 32), jnp.int32),
            jax.ShapeDtypeStruct((T, H // 2), jnp.int32),
        ],
    )(xf, rw, rb2)


def _dispatch_body(x_hbm, p0_hbm, p1_hbm, g0_hbm, g1_hbm, xs_hbm, gs_hbm,
                   rows_v, idx0_v, idx1_v, g0_v, g1_v, sem0, sem1, sem2, sem3):
    wid = lax.axis_index("s") * 2 + lax.axis_index("c")
    base = wid * TPW
    pltpu.sync_copy(x_hbm.at[pl.ds(base, TPW)], rows_v)
    pltpu.sync_copy(p0_hbm.at[pl.ds(base, TPW)], idx0_v)
    pltpu.sync_copy(p1_hbm.at[pl.ds(base, TPW)], idx1_v)
    pltpu.sync_copy(g0_hbm.at[pl.ds(base, TPW)], g0_v)
    pltpu.sync_copy(g1_hbm.at[pl.ds(base, TPW)], g1_v)
    c0 = pltpu.async_copy(rows_v, xs_hbm.at[idx0_v], sem0)
    c1 = pltpu.async_copy(rows_v, xs_hbm.at[idx1_v], sem1)
    c2 = pltpu.async_copy(g0_v, gs_hbm.at[idx0_v], sem2)
    c3 = pltpu.async_copy(g1_v, gs_hbm.at[idx1_v], sem3)
    c0.wait()
    c1.wait()
    c2.wait()
    c3.wait()


@functools.partial(
    pl.kernel,
    out_type=[
        jax.ShapeDtypeStruct((NPAD, H // 2), jnp.int32),
        jax.ShapeDtypeStruct((NPAD,), jnp.float32),
    ],
    mesh=plsc.VectorSubcoreMesh(core_axis_name="c", subcore_axis_name="s"),
    scratch_types=[
        pltpu.VMEM((TPW, H // 2), jnp.int32),
        pltpu.VMEM((TPW,), jnp.int32),
        pltpu.VMEM((TPW,), jnp.int32),
        pltpu.VMEM((TPW,), jnp.float32),
        pltpu.VMEM((TPW,), jnp.float32),
        pltpu.SemaphoreType.DMA,
        pltpu.SemaphoreType.DMA,
        pltpu.SemaphoreType.DMA,
        pltpu.SemaphoreType.DMA,
    ],
)
def _dispatch(x_hbm, p0_hbm, p1_hbm, g0_hbm, g1_hbm, xs_hbm, gs_hbm,
              rows_v, idx0_v, idx1_v, g0_v, g1_v, sem0, sem1, sem2, sem3):
    _dispatch_body(x_hbm, p0_hbm, p1_hbm, g0_hbm, g1_hbm, xs_hbm, gs_hbm,
                   rows_v, idx0_v, idx1_v, g0_v, g1_v, sem0, sem1, sem2, sem3)


def _gemm_body(s_ref, xs_ref, w1_ref, b1_ref, w2_ref, b2_ref, gs_ref, y_ref):
    b = pl.program_id(0)

    @pl.when(b < s_ref[MAXBLKS])
    def _():
        packed = xs_ref[...]
        x_lo = lax.bitcast_convert_type(lax.shift_left(packed, 16), jnp.float32)
        x_hi = lax.bitcast_convert_type(packed & jnp.int32(-65536), jnp.float32)
        xb = jnp.concatenate([x_lo, x_hi], axis=1).astype(jnp.bfloat16)
        w1b = w1_ref[...].astype(jnp.bfloat16)
        h = jnp.dot(xb, w1b, preferred_element_type=jnp.float32)
        h = jnp.maximum(h + b1_ref[...], 0.0).astype(jnp.bfloat16)
        w2b = w2_ref[...].astype(jnp.bfloat16)
        y = jnp.dot(h, w2b, preferred_element_type=jnp.float32) + b2_ref[...]
        y_ref[...] = y * gs_ref[...]


def _gemm(meta, xs, w1, b1r, w2, b2r, gs2):
    grid_spec = pltpu.PrefetchScalarGridSpec(
        num_scalar_prefetch=1,
        grid=(MAXBLKS,),
        in_specs=[
            pl.BlockSpec((BLK, H // 2), lambda b, s: (b, 0)),
            pl.BlockSpec((None, H, F), lambda b, s: (s[b], 0, 0)),
            pl.BlockSpec((None, 1, F), lambda b, s: (s[b], 0, 0)),
            pl.BlockSpec((None, F, H), lambda b, s: (s[b], 0, 0)),
            pl.BlockSpec((None, 1, H), lambda b, s: (s[b], 0, 0)),
            pl.BlockSpec((BLK, 1), lambda b, s: (b, 0)),
        ],
        out_specs=pl.BlockSpec((BLK, H), lambda b, s: (b, 0)),
    )
    return pl.pallas_call(
        _gemm_body,
        grid_spec=grid_spec,
        out_shape=jax.ShapeDtypeStruct((NPAD, H), jnp.float32),
        compiler_params=pltpu.CompilerParams(dimension_semantics=("arbitrary",)),
    )(meta, xs, w1, b1r, w2, b2r, gs2)


def _combine_body(ys_hbm, p0_hbm, p1_hbm, out_hbm, y0_v, y1_v, idx0_v, idx1_v,
                  sem0, sem1):
    wid = lax.axis_index("s") * 2 + lax.axis_index("c")
    for c in range(2):
        base = wid * TPW + c * (TPW // 2)
        pltpu.sync_copy(p0_hbm.at[pl.ds(base, TPW // 2)], idx0_v)
        pltpu.sync_copy(p1_hbm.at[pl.ds(base, TPW // 2)], idx1_v)
        c0 = pltpu.async_copy(ys_hbm.at[idx0_v], y0_v, sem0)
        c1 = pltpu.async_copy(ys_hbm.at[idx1_v], y1_v, sem1)
        c0.wait()
        c1.wait()

        def tbody(t, carry):
            def jbody(j, carry2):
                sl = pl.ds(j * 16, 16)
                y0_v[t, sl] = y0_v[t, sl] + y1_v[t, sl]
                return carry2
            return lax.fori_loop(0, H // 16, jbody, carry)
        lax.fori_loop(0, TPW // 2, tbody, 0)
        pltpu.sync_copy(y0_v, out_hbm.at[pl.ds(base, TPW // 2)])


@functools.partial(
    pl.kernel,
    out_type=jax.ShapeDtypeStruct((T, H), jnp.float32),
    mesh=plsc.VectorSubcoreMesh(core_axis_name="c", subcore_axis_name="s"),
    scratch_types=[
        pltpu.VMEM((TPW // 2, H), jnp.float32),
        pltpu.VMEM((TPW // 2, H), jnp.float32),
        pltpu.VMEM((TPW // 2,), jnp.int32),
        pltpu.VMEM((TPW // 2,), jnp.int32),
        pltpu.SemaphoreType.DMA,
        pltpu.SemaphoreType.DMA,
    ],
)
def _combine(ys_hbm, p0_hbm, p1_hbm, out_hbm, y0_v, y1_v, idx0_v, idx1_v,
             sem0, sem1):
    _combine_body(ys_hbm, p0_hbm, p1_hbm, out_hbm, y0_v, y1_v, idx0_v, idx1_v,
                  sem0, sem1)


def kernel(x, router_w, router_b, w1, b1, w2, b2):
    batch, seq, hidden = x.shape
    xf = x.reshape(T, H)
    pos0, pos1, g0, g1, meta, xb = _router(xf, router_w, router_b.reshape(1, E))
    pos0 = pos0.reshape(T)
    pos1 = pos1.reshape(T)
    xs32, gs = _dispatch(xb, pos0, pos1, g0.reshape(T), g1.reshape(T))
    ys = _gemm(meta.reshape(32), xs32, w1, b1.reshape(E, 1, F), w2,
               b2.reshape(E, 1, H), gs.reshape(NPAD, 1))
    out = _combine(ys, pos0, pos1)
    return out.reshape(batch, seq, hidden)


# speedup vs baseline: 2.1133x; 1.0001x over previous
import functools

import jax
import jax.numpy as jnp
from jax import lax
from jax.experimental import pallas as pl
from jax.experimental.pallas import tpu as pltpu
from jax.experimental.pallas import tpu_sc as plsc

H = 1024
F = 2048
E = 8
T = 2048
BLK = 256
MAXBLKS = T * 2 // BLK + E
NPAD = MAXBLKS * BLK
NW = 32
TPW = T // NW


def _router_body(x_ref, rw_ref, rb_ref, pos0_ref, pos1_ref, g0_ref, g1_ref,
                 meta_ref, xb_ref):
    xf = x_ref[...]
    lo = lax.bitcast_convert_type(
        xf[:, :H // 2].astype(jnp.bfloat16).astype(jnp.float32), jnp.int32)
    hi = lax.bitcast_convert_type(
        xf[:, H // 2:].astype(jnp.bfloat16).astype(jnp.float32), jnp.int32)
    xb_ref[...] = lax.shift_right_logical(lo, 16) | (hi & jnp.int32(-65536))
    logits = jnp.dot(xf, rw_ref[...], preferred_element_type=jnp.float32) + rb_ref[...]
    m = jnp.max(logits, axis=-1, keepdims=True)
    ex = jnp.exp(logits - m)
    probs = ex / jnp.sum(ex, axis=-1, keepdims=True)

    iota_e = lax.broadcasted_iota(jnp.int32, (T, E), 1)
    p0 = jnp.max(probs, axis=-1, keepdims=True)
    ids0 = jnp.min(jnp.where(probs == p0, iota_e, E), axis=-1, keepdims=True)
    sel0 = iota_e == ids0
    masked = jnp.where(sel0, -1e30, probs)
    p1 = jnp.max(masked, axis=-1, keepdims=True)
    ids1 = jnp.min(jnp.where(masked == p1, iota_e, E), axis=-1, keepdims=True)
    sel1 = iota_e == ids1
    denom = p0 + p1
    g0_ref[...] = p0 / denom
    g1_ref[...] = p1 / denom

    c0 = sel0.astype(jnp.float32)
    c1 = sel1.astype(jnp.float32)
    cnt = c0 + c1
    r_i = lax.broadcasted_iota(jnp.int32, (T, T), 0)
    c_i = lax.broadcasted_iota(jnp.int32, (T, T), 1)
    tri = (r_i > c_i).astype(jnp.float32)
    cum_before = jnp.dot(tri, cnt, preferred_element_type=jnp.float32)
    counts = jnp.sum(cnt, axis=0, keepdims=True)
    nblk = jnp.floor((counts + (BLK - 1)) * (1.0 / BLK))
    u_r = lax.broadcasted_iota(jnp.int32, (E, E), 0)
    u_c = lax.broadcasted_iota(jnp.int32, (E, E), 1)
    ut = (u_r <= u_c).astype(jnp.float32)
    s_incl = jnp.dot(nblk, ut, preferred_element_type=jnp.float32)
    pad_off = BLK * (s_incl - nblk)

    pos0f = jnp.sum(jnp.where(sel0, cum_before + pad_off, 0.0), axis=-1, keepdims=True)
    pos1f = jnp.sum(jnp.where(sel1, cum_before + c0 + pad_off, 0.0), axis=-1, keepdims=True)
    pos0_ref[...] = pos0f.astype(jnp.int32)
    pos1_ref[...] = pos1f.astype(jnp.int32)

    iota_b = lax.broadcasted_iota(jnp.int32, (1, 32), 1)
    iota_bf = iota_b.astype(jnp.float32)
    bef = jnp.zeros((1, 32), jnp.float32)
    for e in range(E):
        bef = bef + (s_incl[:, e:e + 1] <= iota_bf).astype(jnp.float32)
    be = jnp.minimum(bef, float(E - 1)).astype(jnp.int32)
    nact = s_incl[:, E - 1:E].astype(jnp.int32)
    meta_ref[...] = jnp.where(iota_b == MAXBLKS, nact, be)


def _router(xf, rw, rb2):
    return pl.pallas_call(
        _router_body,
        out_shape=[
            jax.ShapeDtypeStruct((T, 1), jnp.int32),
            jax.ShapeDtypeStruct((T, 1), jnp.int32),
            jax.ShapeDtypeStruct((T, 1), jnp.float32),
            jax.ShapeDtypeStruct((T, 1), jnp.float32),
            jax.ShapeDtypeStruct((1, 32), jnp.int32),
            jax.ShapeDtypeStruct((T, H // 2), jnp.int32),
        ],
    )(xf, rw, rb2)


def _dispatch_body(x_hbm, p0_hbm, p1_hbm, g0_hbm, g1_hbm, xs_hbm, gs_hbm,
                   rows_v, idx0_v, idx1_v, g0_v, g1_v, sem0, sem1, sem2, sem3):
    wid = lax.axis_index("s") * 2 + lax.axis_index("c")
    base = wid * TPW
    pltpu.sync_copy(x_hbm.at[pl.ds(base, TPW)], rows_v)
    pltpu.sync_copy(p0_hbm.at[pl.ds(base, TPW)], idx0_v)
    pltpu.sync_copy(p1_hbm.at[pl.ds(base, TPW)], idx1_v)
    pltpu.sync_copy(g0_hbm.at[pl.ds(base, TPW)], g0_v)
    pltpu.sync_copy(g1_hbm.at[pl.ds(base, TPW)], g1_v)
    c0 = pltpu.async_copy(rows_v, xs_hbm.at[idx0_v], sem0)
    c1 = pltpu.async_copy(rows_v, xs_hbm.at[idx1_v], sem1)
    c2 = pltpu.async_copy(g0_v, gs_hbm.at[idx0_v], sem2)
    c3 = pltpu.async_copy(g1_v, gs_hbm.at[idx1_v], sem3)
    c0.wait()
    c1.wait()
    c2.wait()
    c3.wait()


@functools.partial(
    pl.kernel,
    out_type=[
        jax.ShapeDtypeStruct((NPAD, H // 2), jnp.int32),
        jax.ShapeDtypeStruct((NPAD,), jnp.float32),
    ],
    mesh=plsc.VectorSubcoreMesh(core_axis_name="c", subcore_axis_name="s"),
    scratch_types=[
        pltpu.VMEM((TPW, H // 2), jnp.int32),
        pltpu.VMEM((TPW,), jnp.int32),
        pltpu.VMEM((TPW,), jnp.int32),
        pltpu.VMEM((TPW,), jnp.float32),
        pltpu.VMEM((TPW,), jnp.float32),
        pltpu.SemaphoreType.DMA,
        pltpu.SemaphoreType.DMA,
        pltpu.SemaphoreType.DMA,
        pltpu.SemaphoreType.DMA,
    ],
)
def _dispatch(x_hbm, p0_hbm, p1_hbm, g0_hbm, g1_hbm, xs_hbm, gs_hbm,
              rows_v, idx0_v, idx1_v, g0_v, g1_v, sem0, sem1, sem2, sem3):
    _dispatch_body(x_hbm, p0_hbm, p1_hbm, g0_hbm, g1_hbm, xs_hbm, gs_hbm,
                   rows_v, idx0_v, idx1_v, g0_v, g1_v, sem0, sem1, sem2, sem3)


def _gemm_body(s_ref, xs_ref, w1_ref, b1_ref, w2_ref, b2_ref, gs_ref, y_ref):
    b = pl.program_id(0)

    @pl.when(b < s_ref[MAXBLKS])
    def _():
        packed = xs_ref[...]
        x_lo = lax.bitcast_convert_type(lax.shift_left(packed, 16), jnp.float32)
        x_hi = lax.bitcast_convert_type(packed & jnp.int32(-65536), jnp.float32)
        xb = jnp.concatenate([x_lo, x_hi], axis=1)
        h = jnp.dot(xb, w1_ref[...], preferred_element_type=jnp.float32,
                    precision=lax.Precision.DEFAULT)
        h = jnp.maximum(h + b1_ref[...], 0.0)
        y = jnp.dot(h, w2_ref[...], preferred_element_type=jnp.float32,
                    precision=lax.Precision.DEFAULT) + b2_ref[...]
        y_ref[...] = y * gs_ref[...]


def _gemm(meta, xs, w1, b1r, w2, b2r, gs2):
    grid_spec = pltpu.PrefetchScalarGridSpec(
        num_scalar_prefetch=1,
        grid=(MAXBLKS,),
        in_specs=[
            pl.BlockSpec((BLK, H // 2), lambda b, s: (b, 0)),
            pl.BlockSpec((None, H, F), lambda b, s: (s[b], 0, 0)),
            pl.BlockSpec((None, 1, F), lambda b, s: (s[b], 0, 0)),
            pl.BlockSpec((None, F, H), lambda b, s: (s[b], 0, 0)),
            pl.BlockSpec((None, 1, H), lambda b, s: (s[b], 0, 0)),
            pl.BlockSpec((BLK, 1), lambda b, s: (b, 0)),
        ],
        out_specs=pl.BlockSpec((BLK, H), lambda b, s: (b, 0)),
    )
    return pl.pallas_call(
        _gemm_body,
        grid_spec=grid_spec,
        out_shape=jax.ShapeDtypeStruct((NPAD, H), jnp.float32),
        compiler_params=pltpu.CompilerParams(dimension_semantics=("arbitrary",)),
    )(meta, xs, w1, b1r, w2, b2r, gs2)


def _combine_body(ys_hbm, p0_hbm, p1_hbm, out_hbm, y0_v, y1_v, idx0_v, idx1_v,
                  sem0, sem1):
    wid = lax.axis_index("s") * 2 + lax.axis_index("c")
    for c in range(2):
        base = wid * TPW + c * (TPW // 2)
        pltpu.sync_copy(p0_hbm.at[pl.ds(base, TPW // 2)], idx0_v)
        pltpu.sync_copy(p1_hbm.at[pl.ds(base, TPW // 2)], idx1_v)
        c0 = pltpu.async_copy(ys_hbm.at[idx0_v], y0_v, sem0)
        c1 = pltpu.async_copy(ys_hbm.at[idx1_v], y1_v, sem1)
        c0.wait()
        c1.wait()

        def tbody(t, carry):
            def jbody(j, carry2):
                sl = pl.ds(j * 16, 16)
                y0_v[t, sl] = y0_v[t, sl] + y1_v[t, sl]
                return carry2
            return lax.fori_loop(0, H // 16, jbody, carry)
        lax.fori_loop(0, TPW // 2, tbody, 0)
        pltpu.sync_copy(y0_v, out_hbm.at[pl.ds(base, TPW // 2)])


@functools.partial(
    pl.kernel,
    out_type=jax.ShapeDtypeStruct((T, H), jnp.float32),
    mesh=plsc.VectorSubcoreMesh(core_axis_name="c", subcore_axis_name="s"),
    scratch_types=[
        pltpu.VMEM((TPW // 2, H), jnp.float32),
        pltpu.VMEM((TPW // 2, H), jnp.float32),
        pltpu.VMEM((TPW // 2,), jnp.int32),
        pltpu.VMEM((TPW // 2,), jnp.int32),
        pltpu.SemaphoreType.DMA,
        pltpu.SemaphoreType.DMA,
    ],
)
def _combine(ys_hbm, p0_hbm, p1_hbm, out_hbm, y0_v, y1_v, idx0_v, idx1_v,
             sem0, sem1):
    _combine_body(ys_hbm, p0_hbm, p1_hbm, out_hbm, y0_v, y1_v, idx0_v, idx1_v,
                  sem0, sem1)


def kernel(x, router_w, router_b, w1, b1, w2, b2):
    batch, seq, hidden = x.shape
    xf = x.reshape(T, H)
    pos0, pos1, g0, g1, meta, xb = _router(xf, router_w, router_b.reshape(1, E))
    pos0 = pos0.reshape(T)
    pos1 = pos1.reshape(T)
    xs32, gs = _dispatch(xb, pos0, pos1, g0.reshape(T), g1.reshape(T))
    ys = _gemm(meta.reshape(32), xs32, w1, b1.reshape(E, 1, F), w2,
               b2.reshape(E, 1, H), gs.reshape(NPAD, 1))
    out = _combine(ys, pos0, pos1)
    return out.reshape(batch, seq, hidden)


# speedup vs baseline: 2.6457x; 1.2519x over previous
import functools

import jax
import jax.numpy as jnp
from jax import lax
from jax.experimental import pallas as pl
from jax.experimental.pallas import tpu as pltpu
from jax.experimental.pallas import tpu_sc as plsc

H = 1024
F = 2048
E = 8
T = 2048
BLK = 256
MAXBLKS = T * 2 // BLK + E
NPAD = MAXBLKS * BLK
NW = 32
TPW = T // NW


def _router_body(x_ref, rw_ref, rb_ref, pos0_ref, pos1_ref, g0_ref, g1_ref,
                 meta_ref, xb_ref):
    xf = x_ref[...]
    lo = lax.bitcast_convert_type(
        xf[:, :H // 2].astype(jnp.bfloat16).astype(jnp.float32), jnp.int32)
    hi = lax.bitcast_convert_type(
        xf[:, H // 2:].astype(jnp.bfloat16).astype(jnp.float32), jnp.int32)
    xb_ref[...] = lax.shift_right_logical(lo, 16) | (hi & jnp.int32(-65536))
    logits = jnp.dot(xf, rw_ref[...], preferred_element_type=jnp.float32) + rb_ref[...]
    m = jnp.max(logits, axis=-1, keepdims=True)
    ex = jnp.exp(logits - m)
    probs = ex / jnp.sum(ex, axis=-1, keepdims=True)

    iota_e = lax.broadcasted_iota(jnp.int32, (T, E), 1)
    p0 = jnp.max(probs, axis=-1, keepdims=True)
    ids0 = jnp.min(jnp.where(probs == p0, iota_e, E), axis=-1, keepdims=True)
    sel0 = iota_e == ids0
    masked = jnp.where(sel0, -1e30, probs)
    p1 = jnp.max(masked, axis=-1, keepdims=True)
    ids1 = jnp.min(jnp.where(masked == p1, iota_e, E), axis=-1, keepdims=True)
    sel1 = iota_e == ids1
    denom = p0 + p1
    g0_ref[...] = p0 / denom
    g1_ref[...] = p1 / denom

    c0 = sel0.astype(jnp.float32)
    c1 = sel1.astype(jnp.float32)
    cnt = c0 + c1
    r_i = lax.broadcasted_iota(jnp.int32, (T, T), 0)
    c_i = lax.broadcasted_iota(jnp.int32, (T, T), 1)
    tri = (r_i > c_i).astype(jnp.float32)
    cum_before = jnp.dot(tri, cnt, preferred_element_type=jnp.float32)
    counts = jnp.sum(cnt, axis=0, keepdims=True)
    nblk = jnp.floor((counts + (BLK - 1)) * (1.0 / BLK))
    u_r = lax.broadcasted_iota(jnp.int32, (E, E), 0)
    u_c = lax.broadcasted_iota(jnp.int32, (E, E), 1)
    ut = (u_r <= u_c).astype(jnp.float32)
    s_incl = jnp.dot(nblk, ut, preferred_element_type=jnp.float32)
    pad_off = BLK * (s_incl - nblk)

    pos0f = jnp.sum(jnp.where(sel0, cum_before + pad_off, 0.0), axis=-1, keepdims=True)
    pos1f = jnp.sum(jnp.where(sel1, cum_before + c0 + pad_off, 0.0), axis=-1, keepdims=True)
    pos0_ref[...] = pos0f.astype(jnp.int32)
    pos1_ref[...] = pos1f.astype(jnp.int32)

    iota_b = lax.broadcasted_iota(jnp.int32, (1, 32), 1)
    iota_bf = iota_b.astype(jnp.float32)
    bef = jnp.zeros((1, 32), jnp.float32)
    for e in range(E):
        bef = bef + (s_incl[:, e:e + 1] <= iota_bf).astype(jnp.float32)
    be = jnp.minimum(bef, float(E - 1)).astype(jnp.int32)
    nact = s_incl[:, E - 1:E].astype(jnp.int32)
    meta_ref[...] = jnp.where(iota_b == MAXBLKS, nact, be)


def _router(xf, rw, rb2):
    return pl.pallas_call(
        _router_body,
        out_shape=[
            jax.ShapeDtypeStruct((T, 1), jnp.int32),
            jax.ShapeDtypeStruct((T, 1), jnp.int32),
            jax.ShapeDtypeStruct((T, 1), jnp.float32),
            jax.ShapeDtypeStruct((T, 1), jnp.float32),
            jax.ShapeDtypeStruct((1, 32), jnp.int32),
            jax.ShapeDtypeStruct((T, H // 2), jnp.int32),
        ],
    )(xf, rw, rb2)


def _dispatch_body(x_hbm, p0_hbm, p1_hbm, xs_hbm,
                   rows_v, idx0_v, idx1_v, sem0, sem1):
    wid = lax.axis_index("s") * 2 + lax.axis_index("c")
    base = wid * TPW
    pltpu.sync_copy(x_hbm.at[pl.ds(base, TPW)], rows_v)
    pltpu.sync_copy(p0_hbm.at[pl.ds(base, TPW)], idx0_v)
    pltpu.sync_copy(p1_hbm.at[pl.ds(base, TPW)], idx1_v)
    c0 = pltpu.async_copy(rows_v, xs_hbm.at[idx0_v], sem0)
    c1 = pltpu.async_copy(rows_v, xs_hbm.at[idx1_v], sem1)
    c0.wait()
    c1.wait()


@functools.partial(
    pl.kernel,
    out_type=jax.ShapeDtypeStruct((NPAD, H // 2), jnp.int32),
    mesh=plsc.VectorSubcoreMesh(core_axis_name="c", subcore_axis_name="s"),
    scratch_types=[
        pltpu.VMEM((TPW, H // 2), jnp.int32),
        pltpu.VMEM((TPW,), jnp.int32),
        pltpu.VMEM((TPW,), jnp.int32),
        pltpu.SemaphoreType.DMA,
        pltpu.SemaphoreType.DMA,
    ],
)
def _dispatch(x_hbm, p0_hbm, p1_hbm, xs_hbm,
              rows_v, idx0_v, idx1_v, sem0, sem1):
    _dispatch_body(x_hbm, p0_hbm, p1_hbm, xs_hbm,
                   rows_v, idx0_v, idx1_v, sem0, sem1)


def _gemm_body(s_ref, xs_ref, w1_ref, b1_ref, w2_ref, b2_ref, y_ref):
    b = pl.program_id(0)

    @pl.when(b < s_ref[MAXBLKS])
    def _():
        packed = xs_ref[...]
        x_lo = lax.bitcast_convert_type(lax.shift_left(packed, 16), jnp.float32)
        x_hi = lax.bitcast_convert_type(packed & jnp.int32(-65536), jnp.float32)
        xb = jnp.concatenate([x_lo, x_hi], axis=1)
        h = jnp.dot(xb, w1_ref[...], preferred_element_type=jnp.float32,
                    precision=lax.Precision.DEFAULT)
        h = jnp.maximum(h + b1_ref[...], 0.0)
        y = jnp.dot(h, w2_ref[...], preferred_element_type=jnp.float32,
                    precision=lax.Precision.DEFAULT) + b2_ref[...]
        y_lo = lax.bitcast_convert_type(
            y[:, :H // 2].astype(jnp.bfloat16).astype(jnp.float32), jnp.int32)
        y_hi = lax.bitcast_convert_type(
            y[:, H // 2:].astype(jnp.bfloat16).astype(jnp.float32), jnp.int32)
        y_ref[...] = lax.shift_right_logical(y_lo, 16) | (y_hi & jnp.int32(-65536))


def _gemm(meta, xs, w1, b1r, w2, b2r):
    grid_spec = pltpu.PrefetchScalarGridSpec(
        num_scalar_prefetch=1,
        grid=(MAXBLKS,),
        in_specs=[
            pl.BlockSpec((BLK, H // 2), lambda b, s: (b, 0)),
            pl.BlockSpec((None, H, F), lambda b, s: (s[b], 0, 0)),
            pl.BlockSpec((None, 1, F), lambda b, s: (s[b], 0, 0)),
            pl.BlockSpec((None, F, H), lambda b, s: (s[b], 0, 0)),
            pl.BlockSpec((None, 1, H), lambda b, s: (s[b], 0, 0)),
        ],
        out_specs=pl.BlockSpec((BLK, H // 2), lambda b, s: (b, 0)),
    )
    return pl.pallas_call(
        _gemm_body,
        grid_spec=grid_spec,
        out_shape=jax.ShapeDtypeStruct((NPAD, H // 2), jnp.int32),
        compiler_params=pltpu.CompilerParams(dimension_semantics=("arbitrary",)),
    )(meta, xs, w1, b1r, w2, b2r)


def _combine_body(ys_hbm, p0_hbm, p1_hbm, g0_hbm, g1_hbm, out_hbm,
                  yp0_v, yp1_v, outb_v, idx0_v, idx1_v, g0_v, g1_v, sem0, sem1):
    wid = lax.axis_index("s") * 2 + lax.axis_index("c")
    pltpu.sync_copy(g0_hbm.at[pl.ds(wid * TPW, TPW)], g0_v.at[pl.ds(0, TPW)])
    pltpu.sync_copy(g1_hbm.at[pl.ds(wid * TPW, TPW)], g1_v.at[pl.ds(0, TPW)])
    mask = jnp.int32(-65536)
    for c in range(2):
        base = wid * TPW + c * (TPW // 2)
        pltpu.sync_copy(p0_hbm.at[pl.ds(base, TPW // 2)], idx0_v)
        pltpu.sync_copy(p1_hbm.at[pl.ds(base, TPW // 2)], idx1_v)
        c0 = pltpu.async_copy(ys_hbm.at[idx0_v], yp0_v, sem0)
        c1 = pltpu.async_copy(ys_hbm.at[idx1_v], yp1_v, sem1)
        c0.wait()
        c1.wait()

        def tbody(t, carry):
            g0s = g0_v[pl.ds(c * (TPW // 2) + t, 16)][0]
            g1s = g1_v[pl.ds(c * (TPW // 2) + t, 16)][0]

            def jbody(j, carry2):
                for u in range(4):
                    off = j * 64 + u * 16
                    sl = pl.ds(off, 16)
                    p0c = yp0_v[t, sl]
                    p1c = yp1_v[t, sl]
                    lo = (lax.bitcast_convert_type(lax.shift_left(p0c, 16), jnp.float32) * g0s
                          + lax.bitcast_convert_type(lax.shift_left(p1c, 16), jnp.float32) * g1s)
                    hi = (lax.bitcast_convert_type(p0c & mask, jnp.float32) * g0s
                          + lax.bitcast_convert_type(p1c & mask, jnp.float32) * g1s)
                    outb_v[t, sl] = lo
                    outb_v[t, pl.ds(H // 2 + off, 16)] = hi
                return carry2
            return lax.fori_loop(0, H // 2 // 64, jbody, carry)
        lax.fori_loop(0, TPW // 2, tbody, 0)
        pltpu.sync_copy(outb_v, out_hbm.at[pl.ds(base, TPW // 2)])


@functools.partial(
    pl.kernel,
    out_type=jax.ShapeDtypeStruct((T, H), jnp.float32),
    mesh=plsc.VectorSubcoreMesh(core_axis_name="c", subcore_axis_name="s"),
    scratch_types=[
        pltpu.VMEM((TPW // 2, H // 2), jnp.int32),
        pltpu.VMEM((TPW // 2, H // 2), jnp.int32),
        pltpu.VMEM((TPW // 2, H), jnp.float32),
        pltpu.VMEM((TPW // 2,), jnp.int32),
        pltpu.VMEM((TPW // 2,), jnp.int32),
        pltpu.VMEM((TPW + 16,), jnp.float32),
        pltpu.VMEM((TPW + 16,), jnp.float32),
        pltpu.SemaphoreType.DMA,
        pltpu.SemaphoreType.DMA,
    ],
)
def _combine(ys_hbm, p0_hbm, p1_hbm, g0_hbm, g1_hbm, out_hbm,
             yp0_v, yp1_v, outb_v, idx0_v, idx1_v, g0_v, g1_v, sem0, sem1):
    _combine_body(ys_hbm, p0_hbm, p1_hbm, g0_hbm, g1_hbm, out_hbm,
                  yp0_v, yp1_v, outb_v, idx0_v, idx1_v, g0_v, g1_v, sem0, sem1)


def kernel(x, router_w, router_b, w1, b1, w2, b2):
    batch, seq, hidden = x.shape
    xf = x.reshape(T, H)
    pos0, pos1, g0, g1, meta, xb = _router(xf, router_w, router_b.reshape(1, E))
    pos0 = pos0.reshape(T)
    pos1 = pos1.reshape(T)
    xs32 = _dispatch(xb, pos0, pos1)
    ys = _gemm(meta.reshape(32), xs32, w1, b1.reshape(E, 1, F), w2,
               b2.reshape(E, 1, H))
    out = _combine(ys, pos0, pos1, g0.reshape(T), g1.reshape(T))
    return out.reshape(batch, seq, hidden)
